# Initial kernel scaffold; baseline (speedup 1.0000x reference)
import jax
import jax.numpy as jnp
from jax.experimental import pallas as pl


def kernel(x):
    raise NotImplementedError("write your pallas kernel here")


# speedup vs baseline: 6.2806x; 6.2806x over previous
import jax
import jax.numpy as jnp
from jax.experimental import pallas as pl

_R = 256
_K = 128


def _cumsum_block(x_ref, o_ref):
    row = jax.lax.broadcasted_iota(jnp.int32, (_K, _K), 0)
    col = jax.lax.broadcasted_iota(jnp.int32, (_K, _K), 1)
    u = (row <= col).astype(jnp.float32)
    n = x_ref.shape[1] // _K
    carry = jnp.zeros((x_ref.shape[0], 1), jnp.float32)
    for j in range(n):
        xj = x_ref[:, j * _K:(j + 1) * _K]
        loc = jnp.dot(xj, u, preferred_element_type=jnp.float32)
        o_ref[:, j * _K:(j + 1) * _K] = loc + carry
        carry = carry + loc[:, _K - 1:_K]


def kernel(x):
    m, c = x.shape
    return pl.pallas_call(
        _cumsum_block,
        grid=(m // _R,),
        in_specs=[pl.BlockSpec((_R, c), lambda i: (i, 0))],
        out_specs=pl.BlockSpec((_R, c), lambda i: (i, 0)),
        out_shape=jax.ShapeDtypeStruct((m, c), x.dtype),
    )(x)


# speedup vs baseline: 6.5473x; 1.0425x over previous
import jax
import jax.numpy as jnp
from jax.experimental import pallas as pl

_R = 512
_K = 128


def _cumsum_block(x_ref, o_ref):
    row = jax.lax.broadcasted_iota(jnp.int32, (_K, _K), 0)
    col = jax.lax.broadcasted_iota(jnp.int32, (_K, _K), 1)
    u = (row <= col).astype(jnp.float32)
    n = x_ref.shape[1] // _K
    carry = jnp.zeros((x_ref.shape[0], 1), jnp.float32)
    for j in range(n):
        xj = x_ref[:, j * _K:(j + 1) * _K]
        loc = jnp.dot(xj, u, preferred_element_type=jnp.float32)
        o_ref[:, j * _K:(j + 1) * _K] = loc + carry
        carry = carry + loc[:, _K - 1:_K]


def kernel(x):
    m, c = x.shape
    return pl.pallas_call(
        _cumsum_block,
        grid=(m // _R,),
        in_specs=[pl.BlockSpec((_R, c), lambda i: (i, 0))],
        out_specs=pl.BlockSpec((_R, c), lambda i: (i, 0)),
        out_shape=jax.ShapeDtypeStruct((m, c), x.dtype),
    )(x)


# speedup vs baseline: 6.5620x; 1.0023x over previous
import jax
import jax.numpy as jnp
from jax.experimental import pallas as pl
from jax.experimental.pallas import tpu as pltpu

_R = 512
_K = 128


def _cumsum_block(x_ref, o_ref):
    row = jax.lax.broadcasted_iota(jnp.int32, (_K, _K), 0)
    col = jax.lax.broadcasted_iota(jnp.int32, (_K, _K), 1)
    u = (row <= col).astype(jnp.float32)
    n = x_ref.shape[1] // _K
    carry = jnp.zeros((x_ref.shape[0], 1), jnp.float32)
    for j in range(n):
        xj = x_ref[:, j * _K:(j + 1) * _K]
        loc = jnp.dot(xj, u, preferred_element_type=jnp.float32)
        o_ref[:, j * _K:(j + 1) * _K] = loc + carry
        carry = carry + loc[:, _K - 1:_K]


def kernel(x):
    m, c = x.shape
    return pl.pallas_call(
        _cumsum_block,
        grid=(m // _R,),
        in_specs=[pl.BlockSpec((_R, c), lambda i: (i, 0))],
        out_specs=pl.BlockSpec((_R, c), lambda i: (i, 0)),
        out_shape=jax.ShapeDtypeStruct((m, c), x.dtype),
        compiler_params=pltpu.CompilerParams(
            vmem_limit_bytes=100 * 1024 * 1024,
        ),
    )(x)
